# baseline (speedup 1.0000x reference)
import jax
import jax.numpy as jnp
from jax.experimental import pallas as pl


def _body(x_ref, m_ref, o_ref):
    o_ref[...] = x_ref[...] * m_ref[...]


def kernel(x, mask):
    R, C = x.shape
    BC = 2048
    return pl.pallas_call(
        _body,
        out_shape=jax.ShapeDtypeStruct((R, C), x.dtype),
        grid=(C // BC,),
        in_specs=[
            pl.BlockSpec((R, BC), lambda j: (0, j)),
            pl.BlockSpec((R, 1), lambda j: (0, 0)),
        ],
        out_specs=pl.BlockSpec((R, BC), lambda j: (0, j)),
    )(x, mask[:, None])
